# Initial kernel scaffold; baseline (speedup 1.0000x reference)
import jax
import jax.numpy as jnp
from jax.experimental import pallas as pl


def kernel(input_tensor):
    raise NotImplementedError("write your pallas kernel here")


# speedup vs baseline: 106.5170x; 106.5170x over previous
import jax
import jax.numpy as jnp
from jax.experimental import pallas as pl
from jax.experimental.pallas import tpu as pltpu

_THRESHOLD = 1.5
_MIN_AREA = 3
_B, _H, _W = 8, 1024, 1024
_BIG = _H * _W


def _roll_r(a):
    return jnp.concatenate([a[:, -1:], a[:, :-1]], axis=1)


def _roll_l(a):
    return jnp.concatenate([a[:, 1:], a[:, :1]], axis=1)


def _roll_d(a):
    return jnp.concatenate([a[-1:, :], a[:-1, :]], axis=0)


def _roll_u(a):
    return jnp.concatenate([a[1:, :], a[:1, :]], axis=0)


def _cluster_kernel(x_ref, out_max_ref, idx_ref):
    x = x_ref[0]
    fg = x >= _THRESHOLD

    def rows_iota():
        return jax.lax.broadcasted_iota(jnp.int32, (_H, _W), 0)

    def cols_iota():
        return jax.lax.broadcasted_iota(jnp.int32, (_H, _W), 1)

    def max3x3(a):
        cols = cols_iota()
        rows = rows_iota()
        h = jnp.maximum(a, jnp.where(cols == 0, 0.0, _roll_r(a)))
        h = jnp.maximum(h, jnp.where(cols == (_W - 1), 0.0, _roll_l(a)))
        m = jnp.maximum(h, jnp.where(rows == 0, 0.0, _roll_d(h)))
        return jnp.maximum(m, jnp.where(rows == (_H - 1), 0.0, _roll_u(h)))

    def min3x3(a):
        cols = cols_iota()
        rows = rows_iota()
        h = jnp.minimum(a, jnp.where(cols == 0, _BIG, _roll_r(a)))
        h = jnp.minimum(h, jnp.where(cols == (_W - 1), _BIG, _roll_l(a)))
        m = jnp.minimum(h, jnp.where(rows == 0, _BIG, _roll_d(h)))
        return jnp.minimum(m, jnp.where(rows == (_H - 1), _BIG, _roll_u(h)))

    def v_body(s):
        v, _ = s
        nv = jnp.where(fg, max3x3(v), 0.0)
        return nv, jnp.any(nv != v)

    v, _ = jax.lax.while_loop(
        lambda s: s[1], v_body, (jnp.where(fg, x, 0.0), jnp.bool_(True))
    )

    c0 = jnp.where(fg & (x == v), rows_iota() * _W + cols_iota(), _BIG)

    def c_body(s):
        c, _ = s
        nc = jnp.where(fg, min3x3(c), _BIG)
        return nc, jnp.any(nc != c)

    c, _ = jax.lax.while_loop(lambda s: s[1], c_body, (c0, jnp.bool_(True)))

    def count_body(i, n):
        dr = i - 3
        rows = rows_iota()
        shifted = pltpu.roll(c, -dr, axis=0)
        rdr = rows + dr
        shifted = jnp.where((rdr >= 0) & (rdr < _H), shifted, -1)
        cols = cols_iota()
        t = (shifted == c).astype(jnp.int32)
        left = shifted
        right = shifted
        for d in range(1, 4):
            left = _roll_l(left)
            right = _roll_r(right)
            t = t + jnp.where(cols >= _W - d, 0, (left == c).astype(jnp.int32))
            t = t + jnp.where(cols < d, 0, (right == c).astype(jnp.int32))
        return n + t

    n = jax.lax.fori_loop(0, 7, count_body, jnp.zeros((_H, _W), jnp.int32))

    valid = fg & (n > _MIN_AREA)
    out_max_ref[0] = jnp.where(valid, v, 0.0)
    idx_ref[0] = jnp.where(valid, c, -1)


@jax.jit
def kernel(input_tensor):
    x = input_tensor.reshape(_B, _H, _W)
    spec = pl.BlockSpec((1, _H, _W), lambda b: (b, 0, 0))
    out_shape = [
        jax.ShapeDtypeStruct((_B, _H, _W), jnp.float32),
        jax.ShapeDtypeStruct((_B, _H, _W), jnp.int32),
    ]
    out_max, idx = pl.pallas_call(
        _cluster_kernel,
        grid=(_B,),
        in_specs=[spec],
        out_specs=[spec, spec],
        out_shape=out_shape,
        compiler_params=pltpu.CompilerParams(
            dimension_semantics=("arbitrary",),
            vmem_limit_bytes=60000 * 1024,
        ),
        name="cluster_above_threshold",
    )(x)
    row = jnp.where(idx < 0, -1, idx // _W)
    col = jnp.where(idx < 0, -1, idx % _W)
    return out_max, row, col
